# baseline (speedup 1.0000x reference)
import functools

import jax
import jax.numpy as jnp
from jax import lax
from jax.experimental import pallas as pl
from jax.experimental.pallas import tpu as pltpu
from jax.experimental.pallas import tpu_sc as plsc

NUM_CLASS = 100000
N_CLS_CTX = 4
CTX_DIM = 512
B = 1024
PREFIX_LEN = 9
SUFFIX_LEN = 64
TOK = PREFIX_LEN + N_CLS_CTX + SUFFIX_LEN
NBCAST = PREFIX_LEN + SUFFIX_LEN
N_ITEMS = 2 * NBCAST

LANES = 16

NC, NS = 2, 16
NW = NC * NS
B_PER_W = B // NW
RB = 8
N_ROUNDS = B_PER_W // RB
REP = 32
HALF_B = B // 2
N_SEG = HALF_B // REP
N_MAIN = 4 * NW
N_LEFT = N_ITEMS - N_MAIN
LSEG = N_LEFT * N_SEG // NW


def _splat_lane(vec, pos):
  ii = lax.iota(jnp.int32, LANES)
  sel = jnp.where(ii == pos, vec, 0)
  return jnp.full((LANES,), jnp.sum(sel), dtype=jnp.int32)


def _sc_assemble(label, tlab, llab, alab, cls_ctx, t3, l3, a3, pre2, suf2):
  mesh = plsc.VectorSubcoreMesh(
      core_axis_name="c", subcore_axis_name="s",
      num_cores=NC, num_subcores=NS)

  @functools.partial(
      pl.kernel,
      out_type=jax.ShapeDtypeStruct((TOK, B, CTX_DIM), jnp.float32),
      mesh=mesh,
      compiler_params=pltpu.CompilerParams(needs_layout_passes=False),
      scratch_types=[
          pltpu.VMEM((B_PER_W,), jnp.int32),
          pltpu.VMEM((B_PER_W,), jnp.int32),
          pltpu.VMEM((B_PER_W,), jnp.int32),
          pltpu.VMEM((B_PER_W,), jnp.int32),
          pltpu.VMEM((REP,), jnp.int32),
          pltpu.VMEM((3, N_CLS_CTX, CTX_DIM), jnp.float32),
          pltpu.VMEM((2, N_CLS_CTX, CTX_DIM), jnp.float32),
          pltpu.VMEM((4, N_CLS_CTX, CTX_DIM), jnp.float32),
          pltpu.VMEM((RB, N_CLS_CTX, CTX_DIM), jnp.float32),
          pltpu.VMEM((RB, N_CLS_CTX, CTX_DIM), jnp.float32),
          pltpu.VMEM((N_CLS_CTX, RB, CTX_DIM), jnp.float32),
          pltpu.VMEM((REP, CTX_DIM), jnp.float32),
          pltpu.VMEM((REP, CTX_DIM), jnp.float32),
          pltpu.SemaphoreType.DMA,
          pltpu.SemaphoreType.DMA,
          pltpu.SemaphoreType.DMA,
          pltpu.SemaphoreType.DMA,
          pltpu.SemaphoreType.DMA,
          pltpu.SemaphoreType.DMA,
      ],
  )
  def k(label_h, tlab_h, llab_h, alab_h, cls_h, t_h, l_h, a_h, pre_h, suf_h,
        out_h,
        lab_v, tl_v, ll_v, al_v, rep_idx, t_v, l_v, a_v,
        cls0, cls1, ctx_t, stage0, stage1,
        sem_g, sem_s, sem_o, sem_p0, sem_p1, sem_lo):
    wid = lax.axis_index("s") * NC + lax.axis_index("c")
    base = wid * B_PER_W
    clsb = (cls0, cls1)
    stages = (stage0, stage1)
    psems = (sem_p0, sem_p1)

    pltpu.sync_copy(t_h, t_v)
    pltpu.sync_copy(l_h, l_v)
    pltpu.sync_copy(a_h, a_v)
    pltpu.sync_copy(label_h.at[pl.ds(base, B_PER_W)], lab_v)
    pltpu.sync_copy(tlab_h.at[pl.ds(base, B_PER_W)], tl_v)
    pltpu.sync_copy(llab_h.at[pl.ds(base, B_PER_W)], ll_v)
    pltpu.sync_copy(alab_h.at[pl.ds(base, B_PER_W)], al_v)
    g_next = pltpu.async_copy(cls_h.at[lab_v.at[pl.ds(0, RB)]], cls0, sem_g)

    def plane_item(item, buf):
      q = item // 2
      halfsel = item - 2 * q
      in_suf = q >= PREFIX_LEN
      p = q + jnp.where(in_suf, TOK - NBCAST, 0)
      qq = q - jnp.where(in_suf, PREFIX_LEN, 0)
      qv = jnp.full((LANES,), qq, dtype=jnp.int32)
      rep_idx[pl.ds(0, LANES)] = qv
      rep_idx[pl.ds(LANES, LANES)] = qv
      @pl.when(jnp.logical_not(in_suf))
      def _():
        pltpu.async_copy(pre_h.at[rep_idx], stages[buf], sem_s).wait()
      @pl.when(in_suf)
      def _():
        pltpu.async_copy(suf_h.at[rep_idx], stages[buf], sem_s).wait()
      last = None
      for kk in range(N_SEG):
        last = pltpu.async_copy(
            stages[buf],
            out_h.at[p, pl.ds(halfsel * HALF_B + kk * REP, REP)],
            psems[buf])
      return last

    state = {"ctx_handle": None, "g_next": g_next}

    def ctx_round(r):
      state["g_next"].wait()
      if r + 1 < N_ROUNDS:
        state["g_next"] = pltpu.async_copy(
            cls_h.at[lab_v.at[pl.ds((r + 1) * RB, RB)]],
            clsb[(r + 1) % 2], sem_g)
      if state["ctx_handle"] is not None:
        for _ in range(N_CLS_CTX):
          state["ctx_handle"].wait()
      cls_rows = clsb[r % 2]
      tl_c = tl_v[pl.ds((r // 2) * LANES, LANES)]
      ll_c = ll_v[pl.ds((r // 2) * LANES, LANES)]
      al_c = al_v[pl.ds((r // 2) * LANES, LANES)]
      roff = (r % 2) * RB

      def row_body(i, carry):
        pos = roff + i
        trow = _splat_lane(tl_c, pos)
        lrow = _splat_lane(ll_c, pos)
        arow = _splat_lane(al_c, pos)
        for c in range(N_CLS_CTX):
          cspl = jnp.full((LANES,), c, dtype=jnp.int32)
          def chunk_body(j, cc, c=c, cspl=cspl):
            o = j * LANES
            s = pl.ds(o, LANES)
            col = lax.iota(jnp.int32, LANES) + o
            mod = (plsc.load_gather(t_v, [trow, cspl, col])
                   + plsc.load_gather(l_v, [lrow, cspl, col])
                   + plsc.load_gather(a_v, [arow, cspl, col]))
            ctx_t[c, i, s] = cls_rows[i, c, s] + mod
            return cc
          lax.fori_loop(0, CTX_DIM // LANES, chunk_body, 0)
        return carry
      lax.fori_loop(0, RB, row_body, 0)
      bb = base + r * RB
      for c in range(N_CLS_CTX):
        state["ctx_handle"] = pltpu.async_copy(
            ctx_t.at[c], out_h.at[PREFIX_LEN + c, pl.ds(bb, RB)], sem_o)

    def leftover_seg(j, sel, i_first):
      s = wid * LSEG + j
      it = N_MAIN + s // N_SEG
      k = s - N_SEG * (s // N_SEG)
      q = it // 2
      halfsel = it - 2 * q
      p = q + jnp.where(q >= PREFIX_LEN, TOK - NBCAST, 0)
      dst = out_h.at[p, pl.ds(halfsel * HALF_B + k * REP, REP)]
      pred = (it == i_first) if sel == 0 else (it != i_first)
      @pl.when(pred)
      def _():
        pltpu.async_copy(stages[sel], dst, sem_lo)

    def gather_stage_row(q, buf):
      in_suf = q >= PREFIX_LEN
      qq = q - jnp.where(in_suf, PREFIX_LEN, 0)
      qv = jnp.full((LANES,), qq, dtype=jnp.int32)
      rep_idx[pl.ds(0, LANES)] = qv
      rep_idx[pl.ds(LANES, LANES)] = qv
      @pl.when(jnp.logical_not(in_suf))
      def _():
        pltpu.async_copy(pre_h.at[rep_idx], stages[buf], sem_s).wait()
      @pl.when(in_suf)
      def _():
        pltpu.async_copy(suf_h.at[rep_idx], stages[buf], sem_s).wait()

    w0 = plane_item(wid, 0)
    w1 = plane_item(wid + NW, 1)
    ctx_round(0)
    for _ in range(N_SEG):
      w0.wait()
    w2 = plane_item(wid + NW * 2, 0)
    ctx_round(1)
    for _ in range(N_SEG):
      w1.wait()
    w3 = plane_item(wid + NW * 3, 1)
    ctx_round(2)
    for _ in range(N_SEG):
      w2.wait()
    s0 = wid * LSEG
    i_first = N_MAIN + s0 // N_SEG
    gather_stage_row(i_first // 2, 0)
    for j in range(LSEG):
      leftover_seg(j, 0, i_first)
    ctx_round(3)
    for _ in range(N_SEG):
      w3.wait()
    i_last = N_MAIN + (s0 + LSEG - 1) // N_SEG
    @pl.when(i_last != i_first)
    def _():
      gather_stage_row(i_last // 2, 1)
    for j in range(LSEG):
      leftover_seg(j, 1, i_first)
    for _ in range(LSEG):
      pltpu.make_async_copy(
          stage0, out_h.at[0, pl.ds(0, REP)], sem_lo).wait()

    for _ in range(N_CLS_CTX):
      state["ctx_handle"].wait()

  return k(label, tlab, llab, alab, cls_ctx, t3, l3, a3, pre2, suf2)


def kernel(label, temperature_label, light_label, angle,
           cls_ctx, temperature_ctx, light_ctx, angle_ctx,
           token_prefix, token_suffix):
  out_t = _sc_assemble(
      label.astype(jnp.int32),
      temperature_label.astype(jnp.int32),
      light_label.astype(jnp.int32),
      angle.astype(jnp.int32),
      cls_ctx, temperature_ctx, light_ctx, angle_ctx,
      token_prefix.reshape(PREFIX_LEN, CTX_DIM),
      token_suffix.reshape(SUFFIX_LEN, CTX_DIM))
  return out_t.transpose(1, 0, 2)


# speedup vs baseline: 1.0005x; 1.0005x over previous
import functools

import jax
import jax.numpy as jnp
from jax import lax
from jax.experimental import pallas as pl
from jax.experimental.pallas import tpu as pltpu
from jax.experimental.pallas import tpu_sc as plsc

NUM_CLASS = 100000
N_CLS_CTX = 4
CTX_DIM = 512
B = 1024
PREFIX_LEN = 9
SUFFIX_LEN = 64
TOK = PREFIX_LEN + N_CLS_CTX + SUFFIX_LEN
NBCAST = PREFIX_LEN + SUFFIX_LEN
N_ITEMS = 2 * NBCAST

LANES = 16

NC, NS = 2, 16
NW = NC * NS
B_PER_W = B // NW
RB = 8
N_ROUNDS = B_PER_W // RB
REP = 32
HALF_B = B // 2
N_SEG = HALF_B // REP
N_MAIN = 4 * NW
N_LEFT = N_ITEMS - N_MAIN
LSEG = N_LEFT * N_SEG // NW


def _splat_lane(vec, pos):
  ii = lax.iota(jnp.int32, LANES)
  sel = jnp.where(ii == pos, vec, 0)
  return jnp.full((LANES,), jnp.sum(sel), dtype=jnp.int32)


def _sc_assemble(label, tlab, llab, alab, cls_ctx, t3, l3, a3, pre2, suf2):
  mesh = plsc.VectorSubcoreMesh(
      core_axis_name="c", subcore_axis_name="s",
      num_cores=NC, num_subcores=NS)

  @functools.partial(
      pl.kernel,
      out_type=jax.ShapeDtypeStruct((TOK, B, CTX_DIM), jnp.float32),
      mesh=mesh,
      compiler_params=pltpu.CompilerParams(needs_layout_passes=False),
      scratch_types=[
          pltpu.VMEM((B_PER_W,), jnp.int32),
          pltpu.VMEM((B_PER_W,), jnp.int32),
          pltpu.VMEM((B_PER_W,), jnp.int32),
          pltpu.VMEM((B_PER_W,), jnp.int32),
          pltpu.VMEM((REP,), jnp.int32),
          pltpu.VMEM((3, N_CLS_CTX, CTX_DIM), jnp.float32),
          pltpu.VMEM((2, N_CLS_CTX, CTX_DIM), jnp.float32),
          pltpu.VMEM((4, N_CLS_CTX, CTX_DIM), jnp.float32),
          pltpu.VMEM((RB, N_CLS_CTX, CTX_DIM), jnp.float32),
          pltpu.VMEM((N_CLS_CTX, RB, CTX_DIM), jnp.float32),
          pltpu.VMEM((REP, CTX_DIM), jnp.float32),
          pltpu.VMEM((REP, CTX_DIM), jnp.float32),
          pltpu.VMEM((REP, CTX_DIM), jnp.float32),
          pltpu.SemaphoreType.DMA,
          pltpu.SemaphoreType.DMA,
          pltpu.SemaphoreType.DMA,
          pltpu.SemaphoreType.DMA,
          pltpu.SemaphoreType.DMA,
          pltpu.SemaphoreType.DMA,
          pltpu.SemaphoreType.DMA,
      ],
  )
  def k(label_h, tlab_h, llab_h, alab_h, cls_h, t_h, l_h, a_h, pre_h, suf_h,
        out_h,
        lab_v, tl_v, ll_v, al_v, rep_idx, t_v, l_v, a_v,
        cls_rows, ctx_t, stage0, stage1, stage2,
        sem_g, sem_s, sem_o, sem_p0, sem_p1, sem_p2, sem_lo):
    wid = lax.axis_index("s") * NC + lax.axis_index("c")
    base = wid * B_PER_W
    stages = (stage0, stage1, stage2)
    psems = (sem_p0, sem_p1, sem_p2)

    pltpu.sync_copy(t_h, t_v)
    pltpu.sync_copy(l_h, l_v)
    pltpu.sync_copy(a_h, a_v)
    pltpu.sync_copy(label_h.at[pl.ds(base, B_PER_W)], lab_v)
    pltpu.sync_copy(tlab_h.at[pl.ds(base, B_PER_W)], tl_v)
    pltpu.sync_copy(llab_h.at[pl.ds(base, B_PER_W)], ll_v)
    pltpu.sync_copy(alab_h.at[pl.ds(base, B_PER_W)], al_v)
    g_next = pltpu.async_copy(
        cls_h.at[lab_v.at[pl.ds(0, RB)]], cls_rows, sem_g)

    def plane_item(item, buf):
      q = item // 2
      halfsel = item - 2 * q
      in_suf = q >= PREFIX_LEN
      p = q + jnp.where(in_suf, TOK - NBCAST, 0)
      qq = q - jnp.where(in_suf, PREFIX_LEN, 0)
      qv = jnp.full((LANES,), qq, dtype=jnp.int32)
      rep_idx[pl.ds(0, LANES)] = qv
      rep_idx[pl.ds(LANES, LANES)] = qv
      @pl.when(jnp.logical_not(in_suf))
      def _():
        pltpu.async_copy(pre_h.at[rep_idx], stages[buf], sem_s).wait()
      @pl.when(in_suf)
      def _():
        pltpu.async_copy(suf_h.at[rep_idx], stages[buf], sem_s).wait()
      last = None
      for kk in range(N_SEG):
        last = pltpu.async_copy(
            stages[buf],
            out_h.at[p, pl.ds(halfsel * HALF_B + kk * REP, REP)],
            psems[buf])
      return last

    state = {"ctx_handle": None, "g_next": g_next}

    def ctx_round(r):
      state["g_next"].wait()
      if state["ctx_handle"] is not None:
        for _ in range(N_CLS_CTX):
          state["ctx_handle"].wait()
      tl_c = tl_v[pl.ds((r // 2) * LANES, LANES)]
      ll_c = ll_v[pl.ds((r // 2) * LANES, LANES)]
      al_c = al_v[pl.ds((r // 2) * LANES, LANES)]
      roff = (r % 2) * RB

      def row_body(i, carry):
        pos = roff + i
        trow = _splat_lane(tl_c, pos)
        lrow = _splat_lane(ll_c, pos)
        arow = _splat_lane(al_c, pos)
        for c in range(N_CLS_CTX):
          cspl = jnp.full((LANES,), c, dtype=jnp.int32)
          def chunk_body(j, cc, c=c, cspl=cspl):
            o = j * LANES
            s = pl.ds(o, LANES)
            col = lax.iota(jnp.int32, LANES) + o
            mod = (plsc.load_gather(t_v, [trow, cspl, col])
                   + plsc.load_gather(l_v, [lrow, cspl, col])
                   + plsc.load_gather(a_v, [arow, cspl, col]))
            ctx_t[c, i, s] = cls_rows[i, c, s] + mod
            return cc
          lax.fori_loop(0, CTX_DIM // LANES, chunk_body, 0)
        return carry
      lax.fori_loop(0, RB, row_body, 0)
      if r + 1 < N_ROUNDS:
        state["g_next"] = pltpu.async_copy(
            cls_h.at[lab_v.at[pl.ds((r + 1) * RB, RB)]], cls_rows, sem_g)
      bb = base + r * RB
      for c in range(N_CLS_CTX):
        state["ctx_handle"] = pltpu.async_copy(
            ctx_t.at[c], out_h.at[PREFIX_LEN + c, pl.ds(bb, RB)], sem_o)

    def leftover_seg(j, sel, i_first, take_first):
      s = wid * LSEG + j
      it = N_MAIN + s // N_SEG
      k = s - N_SEG * (s // N_SEG)
      q = it // 2
      halfsel = it - 2 * q
      p = q + jnp.where(q >= PREFIX_LEN, TOK - NBCAST, 0)
      dst = out_h.at[p, pl.ds(halfsel * HALF_B + k * REP, REP)]
      pred = (it == i_first) if take_first else (it != i_first)
      @pl.when(pred)
      def _():
        pltpu.async_copy(stages[sel], dst, sem_lo)

    def gather_stage_row(q, buf):
      in_suf = q >= PREFIX_LEN
      qq = q - jnp.where(in_suf, PREFIX_LEN, 0)
      qv = jnp.full((LANES,), qq, dtype=jnp.int32)
      rep_idx[pl.ds(0, LANES)] = qv
      rep_idx[pl.ds(LANES, LANES)] = qv
      @pl.when(jnp.logical_not(in_suf))
      def _():
        pltpu.async_copy(pre_h.at[rep_idx], stages[buf], sem_s).wait()
      @pl.when(in_suf)
      def _():
        pltpu.async_copy(suf_h.at[rep_idx], stages[buf], sem_s).wait()

    s0 = wid * LSEG
    i_first = N_MAIN + s0 // N_SEG
    i_last = N_MAIN + (s0 + LSEG - 1) // N_SEG

    w0 = plane_item(wid, 0)
    w1 = plane_item(wid + NW, 1)
    ctx_round(0)
    w2 = plane_item(wid + NW * 2, 2)
    ctx_round(1)
    for _ in range(N_SEG):
      w0.wait()
    w3 = plane_item(wid + NW * 3, 0)
    ctx_round(2)
    for _ in range(N_SEG):
      w1.wait()
    gather_stage_row(i_first // 2, 1)
    for j in range(LSEG):
      leftover_seg(j, 1, i_first, True)
    ctx_round(3)
    for _ in range(N_SEG):
      w2.wait()
    @pl.when(i_last != i_first)
    def _():
      gather_stage_row(i_last // 2, 2)
    for j in range(LSEG):
      leftover_seg(j, 2, i_first, False)
    for _ in range(N_SEG):
      w3.wait()
    for _ in range(LSEG):
      pltpu.make_async_copy(
          stage0, out_h.at[0, pl.ds(0, REP)], sem_lo).wait()

    for _ in range(N_CLS_CTX):
      state["ctx_handle"].wait()

  return k(label, tlab, llab, alab, cls_ctx, t3, l3, a3, pre2, suf2)


def kernel(label, temperature_label, light_label, angle,
           cls_ctx, temperature_ctx, light_ctx, angle_ctx,
           token_prefix, token_suffix):
  out_t = _sc_assemble(
      label.astype(jnp.int32),
      temperature_label.astype(jnp.int32),
      light_label.astype(jnp.int32),
      angle.astype(jnp.int32),
      cls_ctx, temperature_ctx, light_ctx, angle_ctx,
      token_prefix.reshape(PREFIX_LEN, CTX_DIM),
      token_suffix.reshape(SUFFIX_LEN, CTX_DIM))
  return out_t.transpose(1, 0, 2)


# speedup vs baseline: 1.0563x; 1.0557x over previous
import functools

import jax
import jax.numpy as jnp
from jax import lax
from jax.experimental import pallas as pl
from jax.experimental.pallas import tpu as pltpu
from jax.experimental.pallas import tpu_sc as plsc

NUM_CLASS = 100000
N_CLS_CTX = 4
CTX_DIM = 512
B = 1024
PREFIX_LEN = 9
SUFFIX_LEN = 64
TOK = PREFIX_LEN + N_CLS_CTX + SUFFIX_LEN
NBCAST = PREFIX_LEN + SUFFIX_LEN
N_ITEMS = 2 * NBCAST

LANES = 16

NC, NS = 2, 16
NW = NC * NS
B_PER_W = B // NW
RB = 8
N_ROUNDS = B_PER_W // RB
REP = 32
HALF_B = B // 2
N_SEG = HALF_B // REP


def _splat_lane(vec, pos):
  ii = lax.iota(jnp.int32, LANES)
  sel = jnp.where(ii == pos, vec, 0)
  return jnp.full((LANES,), jnp.sum(sel), dtype=jnp.int32)


def _sc_assemble(label, tlab, llab, alab, cls_ctx, t3, l3, a3, pre2, suf2):
  mesh = plsc.VectorSubcoreMesh(
      core_axis_name="c", subcore_axis_name="s",
      num_cores=NC, num_subcores=NS)

  @functools.partial(
      pl.kernel,
      out_type=jax.ShapeDtypeStruct((TOK, B, CTX_DIM), jnp.float32),
      mesh=mesh,
      compiler_params=pltpu.CompilerParams(
          needs_layout_passes=False,
          disable_bounds_checks=True,
          disable_semaphore_checks=True),
      scratch_types=[
          pltpu.VMEM((B_PER_W,), jnp.int32),
          pltpu.VMEM((B_PER_W,), jnp.int32),
          pltpu.VMEM((B_PER_W,), jnp.int32),
          pltpu.VMEM((B_PER_W,), jnp.int32),
          pltpu.VMEM((REP,), jnp.int32),
          pltpu.VMEM((3, N_CLS_CTX, CTX_DIM), jnp.float32),
          pltpu.VMEM((2, N_CLS_CTX, CTX_DIM), jnp.float32),
          pltpu.VMEM((4, N_CLS_CTX, CTX_DIM), jnp.float32),
          pltpu.VMEM((RB, N_CLS_CTX, CTX_DIM), jnp.float32),
          pltpu.VMEM((RB, N_CLS_CTX, CTX_DIM), jnp.float32),
          pltpu.VMEM((N_CLS_CTX, RB, CTX_DIM), jnp.float32),
          pltpu.VMEM((REP, CTX_DIM), jnp.float32),
          pltpu.VMEM((REP, CTX_DIM), jnp.float32),
          pltpu.SemaphoreType.DMA,
          pltpu.SemaphoreType.DMA,
          pltpu.SemaphoreType.DMA,
          pltpu.SemaphoreType.DMA,
          pltpu.SemaphoreType.DMA,
      ],
  )
  def k(label_h, tlab_h, llab_h, alab_h, cls_h, t_h, l_h, a_h, pre_h, suf_h,
        out_h,
        lab_v, tl_v, ll_v, al_v, rep_idx, t_v, l_v, a_v,
        cls0, cls1, ctx_t, stage0, stage1,
        sem_g, sem_s, sem_o, sem_p0, sem_p1):
    wid = lax.axis_index("s") * NC + lax.axis_index("c")
    base = wid * B_PER_W
    clsb = (cls0, cls1)
    stages = (stage0, stage1)
    psems = (sem_p0, sem_p1)

    pltpu.sync_copy(t_h, t_v)
    pltpu.sync_copy(l_h, l_v)
    pltpu.sync_copy(a_h, a_v)
    pltpu.sync_copy(label_h.at[pl.ds(base, B_PER_W)], lab_v)
    pltpu.sync_copy(tlab_h.at[pl.ds(base, B_PER_W)], tl_v)
    pltpu.sync_copy(llab_h.at[pl.ds(base, B_PER_W)], ll_v)
    pltpu.sync_copy(alab_h.at[pl.ds(base, B_PER_W)], al_v)
    g_next = pltpu.async_copy(cls_h.at[lab_v.at[pl.ds(0, RB)]], cls0, sem_g)

    def plane_item(item, buf):
      q = item // 2
      halfsel = item - 2 * q
      in_suf = q >= PREFIX_LEN
      p = q + jnp.where(in_suf, TOK - NBCAST, 0)
      qq = q - jnp.where(in_suf, PREFIX_LEN, 0)
      qv = jnp.full((LANES,), qq, dtype=jnp.int32)
      rep_idx[pl.ds(0, LANES)] = qv
      rep_idx[pl.ds(LANES, LANES)] = qv
      @pl.when(jnp.logical_not(in_suf))
      def _():
        pltpu.async_copy(pre_h.at[rep_idx], stages[buf], sem_s).wait()
      @pl.when(in_suf)
      def _():
        pltpu.async_copy(suf_h.at[rep_idx], stages[buf], sem_s).wait()
      last = None
      for kk in range(N_SEG):
        last = pltpu.async_copy(
            stages[buf],
            out_h.at[p, pl.ds(halfsel * HALF_B + kk * REP, REP)],
            psems[buf])
      return last

    state = {"ctx_handle": None, "g_next": g_next}

    def ctx_round(r):
      state["g_next"].wait()
      if r + 1 < N_ROUNDS:
        state["g_next"] = pltpu.async_copy(
            cls_h.at[lab_v.at[pl.ds((r + 1) * RB, RB)]],
            clsb[(r + 1) % 2], sem_g)
      if state["ctx_handle"] is not None:
        for _ in range(N_CLS_CTX):
          state["ctx_handle"].wait()
      cls_rows = clsb[r % 2]
      tl_c = tl_v[pl.ds((r // 2) * LANES, LANES)]
      ll_c = ll_v[pl.ds((r // 2) * LANES, LANES)]
      al_c = al_v[pl.ds((r // 2) * LANES, LANES)]
      roff = (r % 2) * RB

      def row_body(i, carry):
        pos = roff + i
        trow = _splat_lane(tl_c, pos)
        lrow = _splat_lane(ll_c, pos)
        arow = _splat_lane(al_c, pos)
        for c in range(N_CLS_CTX):
          cspl = jnp.full((LANES,), c, dtype=jnp.int32)
          def chunk_body(j, cc, c=c, cspl=cspl):
            o = j * LANES
            s = pl.ds(o, LANES)
            col = lax.iota(jnp.int32, LANES) + o
            mod = (plsc.load_gather(t_v, [trow, cspl, col])
                   + plsc.load_gather(l_v, [lrow, cspl, col])
                   + plsc.load_gather(a_v, [arow, cspl, col]))
            ctx_t[c, i, s] = cls_rows[i, c, s] + mod
            return cc
          lax.fori_loop(0, CTX_DIM // LANES, chunk_body, 0)
        return carry
      lax.fori_loop(0, RB, row_body, 0)
      bb = base + r * RB
      for c in range(N_CLS_CTX):
        state["ctx_handle"] = pltpu.async_copy(
            ctx_t.at[c], out_h.at[PREFIX_LEN + c, pl.ds(bb, RB)], sem_o)

    w0 = plane_item(wid, 0)
    w1 = plane_item(wid + NW, 1)
    ctx_round(0)
    for _ in range(N_SEG):
      w0.wait()
    w2 = plane_item(wid + NW * 2, 0)
    ctx_round(1)
    for _ in range(N_SEG):
      w1.wait()
    w3 = plane_item(wid + NW * 3, 1)
    ctx_round(2)
    for _ in range(N_SEG):
      w2.wait()
    item4 = wid + NW * 4
    has4 = item4 < N_ITEMS

    @pl.when(has4)
    def _():
      plane_item(item4, 0)
    ctx_round(3)
    for _ in range(N_SEG):
      w3.wait()

    @pl.when(has4)
    def _():
      for kk in range(N_SEG):
        pltpu.make_async_copy(
            stage0, out_h.at[0, pl.ds(kk * REP, REP)], sem_p0).wait()

    for _ in range(N_CLS_CTX):
      state["ctx_handle"].wait()

  return k(label, tlab, llab, alab, cls_ctx, t3, l3, a3, pre2, suf2)


def kernel(label, temperature_label, light_label, angle,
           cls_ctx, temperature_ctx, light_ctx, angle_ctx,
           token_prefix, token_suffix):
  out_t = _sc_assemble(
      label.astype(jnp.int32),
      temperature_label.astype(jnp.int32),
      light_label.astype(jnp.int32),
      angle.astype(jnp.int32),
      cls_ctx, temperature_ctx, light_ctx, angle_ctx,
      token_prefix.reshape(PREFIX_LEN, CTX_DIM),
      token_suffix.reshape(SUFFIX_LEN, CTX_DIM))
  return out_t.transpose(1, 0, 2)
